# baseline (speedup 1.0000x reference)
import jax
import jax.numpy as jnp
from jax import lax
from jax.experimental import pallas as pl
from jax.experimental.pallas import tpu as pltpu

P = 16
B = 64
A = 32
T = 32
MAXD = 4
NDT = 8
NAT = 8
NPOLY = 11
ND = B * MAXD
NATOM = B * A

def _dot16(a, b):
    return jnp.dot(a.astype(jnp.bfloat16), b.astype(jnp.bfloat16),
                   preferred_element_type=jnp.float32)


PPB = 2


def _pose_kernel(xyz_ref, btc_ref, btr_ref, rowtab_ref,
                 coltabT_ref, ptab3_ref, gp_ref,
                 scores_ref, idx_ref):
    for sub in range(PPB):
        _one_pose(xyz_ref[sub], btc_ref[sub], btr_ref[sub],
                  rowtab_ref, coltabT_ref, ptab3_ref, gp_ref,
                  scores_ref, idx_ref, sub)


def _one_pose(xyz, bt_col, bt_row, rowtab_ref,
              coltabT_ref, ptab3_ref, gp_ref, scores_ref, idx_ref, sub):
    f32 = jnp.float32
    i32 = jnp.int32
    ei = lax.broadcasted_iota(i32, (ND, B), 0)
    eb = lax.broadcasted_iota(i32, (ND, B), 1)
    E = ((ei // MAXD) == eb).astype(f32)
    bt256c = _dot16(E, bt_col).astype(i32)

    ri = lax.broadcasted_iota(i32, (ND, T * MAXD), 0)
    rc = lax.broadcasted_iota(i32, (ND, T * MAXD), 1)
    oh128 = ((bt256c == (rc // MAXD)) & ((ri % MAXD) == (rc % MAXD))).astype(f32)
    data = _dot16(oh128, rowtab_ref[...])
    don_local = data[:, 0:1].astype(i32)
    ndon = data[:, 1:2].astype(i32)
    acc_local_c = data[:, 2:3].astype(i32)
    oh_dt = data[:, 3:11]

    slot_col = lax.broadcasted_iota(i32, (ND, 1), 0)
    don_mask = (slot_col % MAXD) < ndon

    tb = lax.broadcasted_iota(i32, (B, ND), 0)
    tj = lax.broadcasted_iota(i32, (B, ND), 1)
    ET = ((tj // MAXD) == tb).astype(f32)
    bt256r = _dot16(bt_row, ET).astype(i32)

    cr = lax.broadcasted_iota(i32, (T * MAXD, ND), 0)
    cj = lax.broadcasted_iota(i32, (T * MAXD, ND), 1)
    oh128T = ((bt256r == (cr // MAXD)) & ((cj % MAXD) == (cr % MAXD))).astype(f32)
    dataT = _dot16(coltabT_ref[...], oh128T)
    acc_local = dataT[0:1, :].astype(i32)
    nacc = dataT[1:2, :].astype(i32)
    oh_atT = dataT[2:10, :]

    slot_row = lax.broadcasted_iota(i32, (1, ND), 1)
    acc_mask = (slot_row % MAXD) < nacc

    M9 = _dot16(E, xyz)
    cc = (lax.broadcasted_iota(i32, (ND, 9 * A), 1) % (3 * A)) // 3
    sel9 = jnp.where(cc == don_local, M9, 0.0)
    gr = lax.broadcasted_iota(i32, (9 * A, 9), 0)
    gc = lax.broadcasted_iota(i32, (9 * A, 9), 1)
    G9 = (gc == ((gr // (3 * A)) * 3 + (gr % 3))).astype(f32)
    don9 = _dot16(sel9, G9)
    don_x = (don9[:, 0:1] + don9[:, 3:4]) + don9[:, 6:7]
    don_y = (don9[:, 1:2] + don9[:, 4:5]) + don9[:, 7:8]
    don_z = (don9[:, 2:3] + don9[:, 5:6]) + don9[:, 8:9]

    sel9a = jnp.where(cc == acc_local_c, M9, 0.0)
    acc9c = _dot16(sel9a, G9)
    acc9 = jnp.transpose(acc9c)
    acc_x = (acc9[0:1, :] + acc9[3:4, :]) + acc9[6:7, :]
    acc_y = (acc9[1:2, :] + acc9[4:5, :]) + acc9[7:8, :]
    acc_z = (acc9[2:3, :] + acc9[5:6, :]) + acc9[8:9, :]

    dx = don_x - acc_x
    dy = don_y - acc_y
    dz = don_z - acc_z
    d2 = ((dx * dx + dy * dy) + dz * dz) + 1e-8
    d = jnp.sqrt(d2)

    mall3 = _dot16(oh_dt, ptab3_ref[...])
    NK = 13 * NAT
    L_all = jnp.concatenate(
        [mall3[:, 0:NK], mall3[:, NK:2 * NK], mall3[:, 2 * NK:3 * NK]],
        axis=0)

    def coeff(k):
        S = _dot16(L_all[:, k * NAT:(k + 1) * NAT], oh_atT)
        return (S[0:ND] + S[ND:2 * ND]) + S[2 * ND:3 * ND]

    val = coeff(0)
    for k in range(1, NPOLY):
        val = val * d + coeff(k)

    pp0 = coeff(NPOLY)
    pp1 = coeff(NPOLY + 1)
    dmin = 0.5 + pp0
    dmax = (dmin + 2.0) + pp1

    mask = don_mask & acc_mask & (d > dmin) & (d < dmax)
    gp = gp_ref[0:1, 0:1]
    energy = jnp.where(mask, val * gp, 0.0)

    s = jnp.sum(energy)
    scores_ref[sub] = jnp.full((1, 128), s, dtype=f32)

    m = jnp.min(energy)
    fi = (lax.broadcasted_iota(i32, (ND, ND), 0) * ND
          + lax.broadcasted_iota(i32, (ND, ND), 1)).astype(f32)
    idxf = jnp.min(jnp.where(energy == m, fi, float(ND * ND)))
    idx_ref[sub] = jnp.full((1, 128), idxf.astype(i32), dtype=i32)


def kernel(coords, block_type, bt_tile_n_donH, bt_tile_n_acc,
           bt_tile_donH_inds, bt_tile_acc_inds, bt_tile_donor_type,
           bt_tile_acceptor_type, pair_params, pair_polynomials,
           global_params):
    f32 = jnp.float32

    def split3(x):
        hi = jax.lax.reduce_precision(x, 8, 7)
        r = x - hi
        mid = jax.lax.reduce_precision(r, 8, 7)
        lo = r - mid
        return hi, mid, lo

    bf16 = jnp.bfloat16
    xyz = coords.reshape(P, B, 3 * A)
    xyz9 = jnp.concatenate(split3(xyz), axis=2).astype(bf16)
    btf = block_type.astype(bf16)
    btc = btf[:, :, None]
    btr = btf[:, None, :]

    dl = bt_tile_donH_inds.astype(f32).reshape(T * MAXD, 1)
    ndn = jnp.broadcast_to(bt_tile_n_donH[:, None].astype(f32),
                           (T, MAXD)).reshape(T * MAXD, 1)
    dt1h = jax.nn.one_hot(bt_tile_donor_type.reshape(-1), NDT, dtype=f32)
    alr = bt_tile_acc_inds.astype(f32).reshape(T * MAXD, 1)
    rowtab = jnp.concatenate([dl, ndn, alr, dt1h], axis=1).astype(bf16)

    al = bt_tile_acc_inds.astype(f32).reshape(T * MAXD, 1)
    nac = jnp.broadcast_to(bt_tile_n_acc[:, None].astype(f32),
                           (T, MAXD)).reshape(T * MAXD, 1)
    at1h = jax.nn.one_hot(bt_tile_acceptor_type.reshape(-1), NAT, dtype=f32)
    coltabT = jnp.concatenate([al, nac, at1h], axis=1).T.astype(bf16)

    pflat = pair_polynomials.transpose(0, 2, 1).reshape(NDT, NPOLY, NAT)
    ppflat = pair_params.transpose(0, 2, 1)[:, :2, :]
    tab = jnp.concatenate([pflat, ppflat], axis=1)
    ptab3 = jnp.concatenate([t.reshape(NDT, 13 * NAT) for t in split3(tab)],
                            axis=1).astype(bf16)

    scores, idx = pl.pallas_call(
        _pose_kernel,
        grid=(P // PPB,),
        compiler_params=pltpu.CompilerParams(
            dimension_semantics=("parallel",)),
        in_specs=[
            pl.BlockSpec((PPB, B, 9 * A), lambda p: (p, 0, 0)),
            pl.BlockSpec((PPB, B, 1), lambda p: (p, 0, 0)),
            pl.BlockSpec((PPB, 1, B), lambda p: (p, 0, 0)),
            pl.BlockSpec((T * MAXD, 11), lambda p: (0, 0)),
            pl.BlockSpec((10, T * MAXD), lambda p: (0, 0)),
            pl.BlockSpec((NDT, 13 * 3 * NAT), lambda p: (0, 0)),
            pl.BlockSpec((1, 5), lambda p: (0, 0)),
        ],
        out_specs=[
            pl.BlockSpec((PPB, 1, 128), lambda p: (p, 0, 0)),
            pl.BlockSpec((PPB, 1, 128), lambda p: (p, 0, 0)),
        ],
        out_shape=[
            jax.ShapeDtypeStruct((P, 1, 128), f32),
            jax.ShapeDtypeStruct((P, 1, 128), jnp.int32),
        ],
    )(xyz9, btc, btr, rowtab, coltabT, ptab3, global_params)

    return scores[:, 0, 0], idx[:, 0, 0]


# speedup vs baseline: 1.0357x; 1.0357x over previous
import jax
import jax.numpy as jnp
from jax import lax
from jax.experimental import pallas as pl
from jax.experimental.pallas import tpu as pltpu

P = 16
B = 64
A = 32
T = 32
MAXD = 4
NDT = 8
NAT = 8
NPOLY = 11
ND = B * MAXD
NATOM = B * A

def _dot16(a, b):
    return jnp.dot(a.astype(jnp.bfloat16), b.astype(jnp.bfloat16),
                   preferred_element_type=jnp.float32)


PPB = 2


def _pose_kernel(xyz_ref, xyzT_ref, btc_ref, btr_ref, rowtab_ref,
                 coltabT_ref, ptab3_ref, gp_ref,
                 scores_ref, idx_ref):
    for sub in range(PPB):
        _one_pose(xyz_ref[sub], xyzT_ref[sub], btc_ref[sub], btr_ref[sub],
                  rowtab_ref, coltabT_ref, ptab3_ref, gp_ref,
                  scores_ref, idx_ref, sub)


def _one_pose(xyz, xyzT, bt_col, bt_row, rowtab_ref,
              coltabT_ref, ptab3_ref, gp_ref, scores_ref, idx_ref, sub):
    f32 = jnp.float32
    i32 = jnp.int32
    ei = lax.broadcasted_iota(i32, (ND, B), 0)
    eb = lax.broadcasted_iota(i32, (ND, B), 1)
    E = ((ei // MAXD) == eb).astype(f32)
    bt256c = _dot16(E, bt_col).astype(i32)

    ri = lax.broadcasted_iota(i32, (ND, T * MAXD), 0)
    rc = lax.broadcasted_iota(i32, (ND, T * MAXD), 1)
    oh128 = ((bt256c == (rc // MAXD)) & ((ri % MAXD) == (rc % MAXD))).astype(f32)
    data = _dot16(oh128, rowtab_ref[...])
    don_local = data[:, 0:1].astype(i32)
    ndon = data[:, 1:2].astype(i32)
    oh_dt = data[:, 2:10]

    slot_col = lax.broadcasted_iota(i32, (ND, 1), 0)
    don_mask = (slot_col % MAXD) < ndon

    tb = lax.broadcasted_iota(i32, (B, ND), 0)
    tj = lax.broadcasted_iota(i32, (B, ND), 1)
    ET = ((tj // MAXD) == tb).astype(f32)
    bt256r = _dot16(bt_row, ET).astype(i32)

    cr = lax.broadcasted_iota(i32, (T * MAXD, ND), 0)
    cj = lax.broadcasted_iota(i32, (T * MAXD, ND), 1)
    oh128T = ((bt256r == (cr // MAXD)) & ((cj % MAXD) == (cr % MAXD))).astype(f32)
    dataT = _dot16(coltabT_ref[...], oh128T)
    acc_local = dataT[0:1, :].astype(i32)
    nacc = dataT[1:2, :].astype(i32)
    oh_atT = dataT[2:10, :]

    slot_row = lax.broadcasted_iota(i32, (1, ND), 1)
    acc_mask = (slot_row % MAXD) < nacc

    M9 = _dot16(E, xyz)
    cc = lax.broadcasted_iota(i32, (ND, 9 * A), 1) % A
    sel9 = jnp.where(cc == don_local, M9, 0.0)
    gr = lax.broadcasted_iota(i32, (9 * A, 9), 0)
    gc = lax.broadcasted_iota(i32, (9 * A, 9), 1)
    G9 = (gc == ((gr // (3 * A)) * 3 + (gr % (3 * A)) // A)).astype(f32)
    don9 = _dot16(sel9, G9)
    don_x = (don9[:, 0:1] + don9[:, 3:4]) + don9[:, 6:7]
    don_y = (don9[:, 1:2] + don9[:, 4:5]) + don9[:, 7:8]
    don_z = (don9[:, 2:3] + don9[:, 5:6]) + don9[:, 8:9]

    S9 = _dot16(xyzT, ET)
    rr = lax.broadcasted_iota(i32, (9 * A, ND), 0) % A
    selT9 = jnp.where(rr == acc_local, S9, 0.0)
    jr = lax.broadcasted_iota(i32, (9, 9 * A), 0)
    jc = lax.broadcasted_iota(i32, (9, 9 * A), 1)
    G9T = (jr == ((jc // (3 * A)) * 3 + (jc % (3 * A)) // A)).astype(f32)
    acc9 = _dot16(G9T, selT9)
    acc_x = (acc9[0:1, :] + acc9[3:4, :]) + acc9[6:7, :]
    acc_y = (acc9[1:2, :] + acc9[4:5, :]) + acc9[7:8, :]
    acc_z = (acc9[2:3, :] + acc9[5:6, :]) + acc9[8:9, :]

    dx = don_x - acc_x
    dy = don_y - acc_y
    dz = don_z - acc_z
    d2 = ((dx * dx + dy * dy) + dz * dz) + 1e-8
    d = jnp.sqrt(d2)

    mall3 = _dot16(oh_dt, ptab3_ref[...])
    NK = 13 * NAT
    L_all = jnp.concatenate(
        [mall3[:, 0:NK], mall3[:, NK:2 * NK], mall3[:, 2 * NK:3 * NK]],
        axis=0)

    def coeff(k):
        S = _dot16(L_all[:, k * NAT:(k + 1) * NAT], oh_atT)
        return (S[0:ND] + S[ND:2 * ND]) + S[2 * ND:3 * ND]

    val = coeff(0)
    for k in range(1, NPOLY):
        val = val * d + coeff(k)

    pp0 = coeff(NPOLY)
    pp1 = coeff(NPOLY + 1)
    dmin = 0.5 + pp0
    dmax = (dmin + 2.0) + pp1

    mask = don_mask & acc_mask & (d > dmin) & (d < dmax)
    gp = gp_ref[0:1, 0:1]
    energy = jnp.where(mask, val * gp, 0.0)

    s = jnp.sum(energy)
    scores_ref[sub] = jnp.full((1, 128), s, dtype=f32)

    m = jnp.min(energy)
    fi = (lax.broadcasted_iota(i32, (ND, ND), 0) * ND
          + lax.broadcasted_iota(i32, (ND, ND), 1)).astype(f32)
    idxf = jnp.min(jnp.where(energy == m, fi, float(ND * ND)))
    idx_ref[sub] = jnp.full((1, 128), idxf.astype(i32), dtype=i32)


def kernel(coords, block_type, bt_tile_n_donH, bt_tile_n_acc,
           bt_tile_donH_inds, bt_tile_acc_inds, bt_tile_donor_type,
           bt_tile_acceptor_type, pair_params, pair_polynomials,
           global_params):
    f32 = jnp.float32

    def split3(x):
        hi = jax.lax.reduce_precision(x, 8, 7)
        r = x - hi
        mid = jax.lax.reduce_precision(r, 8, 7)
        lo = r - mid
        return hi, mid, lo

    bf16 = jnp.bfloat16
    xyz = coords.reshape(P, B, A, 3).transpose(0, 1, 3, 2).reshape(P, B, 3 * A)
    xyz9 = jnp.concatenate(split3(xyz), axis=2).astype(bf16)
    xyzT9 = jnp.transpose(xyz9, (0, 2, 1))
    btf = block_type.astype(bf16)
    btc = btf[:, :, None]
    btr = btf[:, None, :]

    dl = bt_tile_donH_inds.astype(f32).reshape(T * MAXD, 1)
    ndn = jnp.broadcast_to(bt_tile_n_donH[:, None].astype(f32),
                           (T, MAXD)).reshape(T * MAXD, 1)
    dt1h = jax.nn.one_hot(bt_tile_donor_type.reshape(-1), NDT, dtype=f32)
    rowtab = jnp.concatenate([dl, ndn, dt1h], axis=1).astype(bf16)

    al = bt_tile_acc_inds.astype(f32).reshape(T * MAXD, 1)
    nac = jnp.broadcast_to(bt_tile_n_acc[:, None].astype(f32),
                           (T, MAXD)).reshape(T * MAXD, 1)
    at1h = jax.nn.one_hot(bt_tile_acceptor_type.reshape(-1), NAT, dtype=f32)
    coltabT = jnp.concatenate([al, nac, at1h], axis=1).T.astype(bf16)

    pflat = pair_polynomials.transpose(0, 2, 1).reshape(NDT, NPOLY, NAT)
    ppflat = pair_params.transpose(0, 2, 1)[:, :2, :]
    tab = jnp.concatenate([pflat, ppflat], axis=1)
    ptab3 = jnp.concatenate([t.reshape(NDT, 13 * NAT) for t in split3(tab)],
                            axis=1).astype(bf16)

    scores, idx = pl.pallas_call(
        _pose_kernel,
        grid=(P // PPB,),
        compiler_params=pltpu.CompilerParams(
            dimension_semantics=("parallel",)),
        in_specs=[
            pl.BlockSpec((PPB, B, 9 * A), lambda p: (p, 0, 0)),
            pl.BlockSpec((PPB, 9 * A, B), lambda p: (p, 0, 0)),
            pl.BlockSpec((PPB, B, 1), lambda p: (p, 0, 0)),
            pl.BlockSpec((PPB, 1, B), lambda p: (p, 0, 0)),
            pl.BlockSpec((T * MAXD, 10), lambda p: (0, 0)),
            pl.BlockSpec((10, T * MAXD), lambda p: (0, 0)),
            pl.BlockSpec((NDT, 13 * 3 * NAT), lambda p: (0, 0)),
            pl.BlockSpec((1, 5), lambda p: (0, 0)),
        ],
        out_specs=[
            pl.BlockSpec((PPB, 1, 128), lambda p: (p, 0, 0)),
            pl.BlockSpec((PPB, 1, 128), lambda p: (p, 0, 0)),
        ],
        out_shape=[
            jax.ShapeDtypeStruct((P, 1, 128), f32),
            jax.ShapeDtypeStruct((P, 1, 128), jnp.int32),
        ],
    )(xyz9, xyzT9, btc, btr, rowtab, coltabT, ptab3, global_params)

    return scores[:, 0, 0], idx[:, 0, 0]
